# baseline (speedup 1.0000x reference)
import numpy as np
import jax
import jax.numpy as jnp
from jax.experimental import pallas as pl
from jax.experimental.pallas import tpu as pltpu

_INTERPRET = False

B = 16
C, H, W = 3, 64, 64
K = 16
L = K * K * C
HIDDEN = 512
NUM_CLASSES = 1000

H1, W1 = 66, 72
N1 = H1 * W1
BASE1 = W1 + 1
M1 = N1 - 2 * BASE1
H2, W2 = 18, 24
N2 = H2 * W2
BASE2 = W2 + 1
M2 = N2 - 2 * BASE2
H3, W3 = 10, 16
N3 = H3 * W3
BASE3 = W3 + 1
M3 = N3 - 2 * BASE3


def _interior_mask(hp, wp, hi, wi):
    base = wp + 1
    m = hp * wp - 2 * base
    q = np.arange(m) + base
    h = q // wp - 1
    w = q % wp - 1
    ok = (h >= 0) & (h < hi) & (w >= 0) & (w < wi)
    return ok.astype(np.float32)[:, None]

_MASK1 = _interior_mask(H1, W1, 64, 64)
_MASK2 = _interior_mask(H2, W2, 16, 16)
_MASK3 = _interior_mask(H3, W3, 8, 8)

_IDXNP = np.array(list(np.ndindex((K, K, C))), dtype=np.int64)[:, (2, 0, 1)]
_PY = (_IDXNP[:, 1].astype(np.float32) / K)[None, :]
_PX = (_IDXNP[:, 2].astype(np.float32) / K)[None, :]
_CHB = (_IDXNP[:, 0].astype(np.float32) * H)[None, :]
_PERM = np.empty((L,), dtype=np.int32)
for _co in range(C):
    for _ho in range(K):
        for _wo in range(K):
            _PERM[_co * K * K + _ho * K + _wo] = _ho * K * C + _wo * C + _co


def _conv9(x_frame, w9, m_rows, base):
    acc = None
    for dy in range(3):
        for dx in range(3):
            st = base + (dy - 1) * (base - 1) + (dx - 1)
            part = jnp.dot(x_frame[st:st + m_rows, :], w9[dy * 3 + dx],
                           preferred_element_type=jnp.float32)
            acc = part if acc is None else acc + part
    return acc


def _embed(y, base, cout):
    z = jnp.zeros((base, cout), dtype=jnp.float32)
    return jnp.concatenate([z, y, z], axis=0)


def _kernel_a(x_ref, w1_ref, b1_ref, w2_ref, b2_ref, m_ref, o_ref):
    x = x_ref[0]
    mask = m_ref[...]
    y1 = jnp.maximum(_conv9(x, w1_ref[...], M1, BASE1) + b1_ref[...], 0.) * mask
    f2 = _embed(y1, BASE1, 64)
    y2 = jnp.maximum(_conv9(f2, w2_ref[...], M1, BASE1) + b2_ref[...], 0.) * mask
    f3 = _embed(y2, BASE1, 64).reshape(H1, W1, 64)
    inter = f3[1:65, 1:65, :]
    t = inter.reshape(1024, 4, 64).max(axis=1)
    t = t.reshape(16, 4, 16, 64).max(axis=1)
    o_ref[0] = t.reshape(256, 64)


def _kernel_b(x_ref, w3_ref, b3_ref, w4_ref, b4_ref, w5_ref, b5_ref,
              w6_ref, b6_ref, m2_ref, m3_ref, o_ref):
    x = x_ref[0]
    mask2 = m2_ref[...]
    y3 = jnp.maximum(_conv9(x, w3_ref[...], M2, BASE2) + b3_ref[...], 0.) * mask2
    f4 = _embed(y3, BASE2, 128)
    y4 = jnp.maximum(_conv9(f4, w4_ref[...], M2, BASE2) + b4_ref[...], 0.) * mask2
    f5 = _embed(y4, BASE2, 128).reshape(H2, W2, 128)
    inter = f5[1:17, 1:17, :]
    t = inter.reshape(128, 2, 128).max(axis=1)
    t = t.reshape(8, 2, 8, 128).max(axis=1)
    zc = jnp.zeros((8, 1, 128), dtype=jnp.float32)
    zc7 = jnp.zeros((8, 7, 128), dtype=jnp.float32)
    t = jnp.concatenate([zc, t, zc7], axis=1)
    zr = jnp.zeros((1, 16, 128), dtype=jnp.float32)
    f6 = jnp.concatenate([zr, t, zr], axis=0).reshape(N3, 128)
    mask3 = m3_ref[...]
    y5 = jnp.maximum(_conv9(f6, w5_ref[...], M3, BASE3) + b5_ref[...], 0.) * mask3
    f7 = _embed(y5, BASE3, 512)
    y6 = jnp.maximum(_conv9(f7, w6_ref[...], M3, BASE3) + b6_ref[...], 0.) * mask3
    f8 = _embed(y6, BASE3, 512).reshape(H3, W3, 512)
    inter3 = f8[1:9, 1:9, :].reshape(64, 512)
    o_ref[0] = inter3.T


def _kernel_c(x_ref, pw1_ref, pb1_ref, pw2_ref, pb2_ref, pw3_ref, pb3_ref,
              o_ref, acc_ref):
    i = pl.program_id(0)

    @pl.when(i == 0)
    def _():
        acc_ref[...] = jnp.zeros_like(acc_ref)

    acc_ref[...] += jnp.dot(x_ref[...], pw1_ref[...],
                            preferred_element_type=jnp.float32)

    @pl.when(i == pl.num_programs(0) - 1)
    def _():
        h1 = jnp.maximum(acc_ref[...] + pb1_ref[...], 0.)
        h2 = jnp.maximum(jnp.dot(h1, pw2_ref[...],
                                 preferred_element_type=jnp.float32)
                         + pb2_ref[...], 0.)
        o_ref[...] = jnp.dot(h2, pw3_ref[...],
                             preferred_element_type=jnp.float32) + pb3_ref[...]


def _kernel_d(bb_ref, img_ref, py_ref, px_ref, chb_ref, hs_ref, hv_ref, o_ref):
    bb = bb_ref[0]
    ymin = bb[0:1, 0:1] - 1.0
    ymax = bb[0:1, 1:2] + 1.0
    xmin = bb[0:1, 2:3] - 1.0
    xmax = bb[0:1, 3:4] + 1.0
    my = jax.nn.sigmoid(py_ref[...] * (ymax - ymin) + ymin) * (H - 1)
    mx = jax.nn.sigmoid(px_ref[...] * (xmax - xmin) + xmin) * (W - 1)
    y0 = jnp.floor(my)
    x0 = jnp.floor(mx)
    hs = hs_ref[...]
    sp = jnp.maximum(hs, 0.) + jnp.log1p(jnp.exp(-jnp.abs(hs)))
    sig = sp + 1e-4
    imgT = img_ref[0]
    corners = [(0., 0.), (0., 1.), (1., 0.), (1., 1.)]
    es, gs = [], []
    for dy, dx in corners:
        yc = y0 + dy
        xc = x0 + dx
        d2 = (yc - my) ** 2 + (xc - mx) ** 2
        es.append(jnp.exp(-d2 / (2.0 * sig * sig)))
        yi = jnp.clip(yc, 0., H - 1.)
        xi = jnp.clip(xc, 0., W - 1.)
        rid = (chb_ref[...] + yi).astype(jnp.int32)
        oh = (jax.lax.broadcasted_iota(jnp.int32, (3 * H, L), 0)
              == rid).astype(jnp.float32)
        g = jnp.dot(imgT, oh, preferred_element_type=jnp.float32, precision=jax.lax.Precision.HIGHEST)
        ohx = (jax.lax.broadcasted_iota(jnp.int32, (W, L), 0)
               == xi.astype(jnp.int32)).astype(jnp.float32)
        gs.append(jnp.sum(g * ohx, axis=0, keepdims=True))
    denom = es[0] + es[1] + es[2] + es[3] + 1e-9
    point = (gs[0] * es[0] + gs[1] * es[1] + gs[2] * es[2] + gs[3] * es[3])
    point = point / denom * hv_ref[...]
    o_ref[0] = point


def _kernel_e(x_ref, hb_ref, w1_ref, b1_ref, w2_ref, b2_ref, o_ref):
    x = x_ref[...] + hb_ref[...]
    h1 = jnp.maximum(jnp.dot(x, w1_ref[...],
                             preferred_element_type=jnp.float32) + b1_ref[...], 0.)
    lg = jnp.dot(h1, w2_ref[...], preferred_element_type=jnp.float32) + b2_ref[...]
    m = jnp.max(lg, axis=1, keepdims=True)
    e = jnp.exp(lg - m)
    o_ref[...] = e / jnp.sum(e, axis=1, keepdims=True)


def _wt(cw, cin, cout):
    return cw.transpose(2, 3, 1, 0).reshape(9, cin, cout)


def kernel(image, cw1, cb1, cw2, cb2, cw3, cb3, cw4, cb4, cw5, cb5, cw6, cb6,
           pw1, pb1, pw2, pb2, pw3, pb3, h_sigmas, h_values, h_bias,
           lin1_w, lin1_b, lin2_w, lin2_b):
    f32 = jnp.float32
    b = image.shape[0]

    xp = jnp.pad(image.transpose(0, 2, 3, 1), ((0, 0), (1, 1), (1, 7), (0, 0)))
    xp = xp.reshape(b, N1, C)
    a_out = pl.pallas_call(
        _kernel_a,
        grid=(b,),
        in_specs=[
            pl.BlockSpec((1, N1, C), lambda i: (i, 0, 0)),
            pl.BlockSpec((9, C, 64), lambda i: (0, 0, 0)),
            pl.BlockSpec((1, 64), lambda i: (0, 0)),
            pl.BlockSpec((9, 64, 64), lambda i: (0, 0, 0)),
            pl.BlockSpec((1, 64), lambda i: (0, 0)),
            pl.BlockSpec((M1, 1), lambda i: (0, 0)),
        ],
        out_specs=pl.BlockSpec((1, 256, 64), lambda i: (i, 0, 0)),
        out_shape=jax.ShapeDtypeStruct((b, 256, 64), f32),
        interpret=_INTERPRET,
    )(xp, _wt(cw1, C, 64), cb1.reshape(1, 64), _wt(cw2, 64, 64),
      cb2.reshape(1, 64), jnp.asarray(_MASK1))

    xb = jnp.pad(a_out.reshape(b, 16, 16, 64), ((0, 0), (1, 1), (1, 7), (0, 0)))
    xb = xb.reshape(b, N2, 64)
    b_out = pl.pallas_call(
        _kernel_b,
        grid=(b,),
        in_specs=[
            pl.BlockSpec((1, N2, 64), lambda i: (i, 0, 0)),
            pl.BlockSpec((9, 64, 128), lambda i: (0, 0, 0)),
            pl.BlockSpec((1, 128), lambda i: (0, 0)),
            pl.BlockSpec((9, 128, 128), lambda i: (0, 0, 0)),
            pl.BlockSpec((1, 128), lambda i: (0, 0)),
            pl.BlockSpec((9, 128, 512), lambda i: (0, 0, 0)),
            pl.BlockSpec((1, 512), lambda i: (0, 0)),
            pl.BlockSpec((9, 512, 512), lambda i: (0, 0, 0)),
            pl.BlockSpec((1, 512), lambda i: (0, 0)),
            pl.BlockSpec((M2, 1), lambda i: (0, 0)),
            pl.BlockSpec((M3, 1), lambda i: (0, 0)),
        ],
        out_specs=pl.BlockSpec((1, 512, 64), lambda i: (i, 0, 0)),
        out_shape=jax.ShapeDtypeStruct((b, 512, 64), f32),
        interpret=_INTERPRET,
    )(xb, _wt(cw3, 64, 128), cb3.reshape(1, 128), _wt(cw4, 128, 128),
      cb4.reshape(1, 128), _wt(cw5, 128, 512), cb5.reshape(1, 512),
      _wt(cw6, 512, 512), cb6.reshape(1, 512),
      jnp.asarray(_MASK2), jnp.asarray(_MASK3))

    flat = b_out.reshape(b, 32768)
    kc = 16
    blk = 32768 // kc
    bbox = pl.pallas_call(
        _kernel_c,
        grid=(kc,),
        in_specs=[
            pl.BlockSpec((b, blk), lambda i: (0, i)),
            pl.BlockSpec((blk, 512), lambda i: (i, 0)),
            pl.BlockSpec((1, 512), lambda i: (0, 0)),
            pl.BlockSpec((512, 512), lambda i: (0, 0)),
            pl.BlockSpec((1, 512), lambda i: (0, 0)),
            pl.BlockSpec((512, 4), lambda i: (0, 0)),
            pl.BlockSpec((1, 4), lambda i: (0, 0)),
        ],
        out_specs=pl.BlockSpec((b, 4), lambda i: (0, 0)),
        out_shape=jax.ShapeDtypeStruct((b, 4), f32),
        scratch_shapes=[pltpu.VMEM((b, 512), f32)],
        compiler_params=pltpu.CompilerParams(
            dimension_semantics=("arbitrary",)),
        interpret=_INTERPRET,
    )(flat, pw1, pb1.reshape(1, 512), pw2, pb2.reshape(1, 512),
      pw3, pb3.reshape(1, 4))

    imgT = image.transpose(0, 3, 1, 2).reshape(b, W, C * H)
    point = pl.pallas_call(
        _kernel_d,
        grid=(b,),
        in_specs=[
            pl.BlockSpec((1, 1, 4), lambda i: (i, 0, 0)),
            pl.BlockSpec((1, W, C * H), lambda i: (i, 0, 0)),
            pl.BlockSpec((1, L), lambda i: (0, 0)),
            pl.BlockSpec((1, L), lambda i: (0, 0)),
            pl.BlockSpec((1, L), lambda i: (0, 0)),
            pl.BlockSpec((1, L), lambda i: (0, 0)),
            pl.BlockSpec((1, L), lambda i: (0, 0)),
        ],
        out_specs=pl.BlockSpec((1, 1, L), lambda i: (i, 0, 0)),
        out_shape=jax.ShapeDtypeStruct((b, 1, L), f32),
        interpret=_INTERPRET,
    )(bbox.reshape(b, 1, 4), imgT, jnp.asarray(_PY), jnp.asarray(_PX), jnp.asarray(_CHB),
      (h_sigmas * 0.1 + 0.01).reshape(1, L), h_values.reshape(1, L))

    flat_g = point.reshape(b, L)[:, jnp.asarray(_PERM)]
    probs = pl.pallas_call(
        _kernel_e,
        out_shape=jax.ShapeDtypeStruct((b, NUM_CLASSES), f32),
        interpret=_INTERPRET,
    )(flat_g, h_bias.reshape(1, L), lin1_w, lin1_b.reshape(1, HIDDEN),
      lin2_w, lin2_b.reshape(1, NUM_CLASSES))
    return probs


# speedup vs baseline: 1.1495x; 1.1495x over previous
import numpy as np
import jax
import jax.numpy as jnp
from jax.experimental import pallas as pl
from jax.experimental.pallas import tpu as pltpu

B = 16
C, H, W = 3, 64, 64
K = 16
L = K * K * C
HIDDEN = 512
NUM_CLASSES = 1000

H1, W1 = 66, 72
N1 = H1 * W1
BASE1 = W1 + 1
M1 = N1 - 2 * BASE1
H2, W2 = 18, 24
N2 = H2 * W2
BASE2 = W2 + 1
M2 = N2 - 2 * BASE2
H3, W3 = 10, 16
N3 = H3 * W3
BASE3 = W3 + 1
M3 = N3 - 2 * BASE3


def _interior_mask(hp, wp, hi, wi):
    base = wp + 1
    m = hp * wp - 2 * base
    q = np.arange(m) + base
    h = q // wp - 1
    w = q % wp - 1
    ok = (h >= 0) & (h < hi) & (w >= 0) & (w < wi)
    return ok.astype(np.float32)[:, None]

_MASK1 = _interior_mask(H1, W1, 64, 64)
_MASK2 = _interior_mask(H2, W2, 16, 16)
_MASK3 = _interior_mask(H3, W3, 8, 8)

_IDXNP = np.array(list(np.ndindex((K, K, C))), dtype=np.int64)[:, (2, 0, 1)]
_PY = (_IDXNP[:, 1].astype(np.float32) / K)[None, :]
_PX = (_IDXNP[:, 2].astype(np.float32) / K)[None, :]
_CHB = (_IDXNP[:, 0].astype(np.float32) * H)[None, :]
_PERM = np.empty((L,), dtype=np.int32)
for _co in range(C):
    for _ho in range(K):
        for _wo in range(K):
            _PERM[_co * K * K + _ho * K + _wo] = _ho * K * C + _wo * C + _co


def _conv9(x_frame, w9, m_rows, base):
    cols = []
    for dy in range(3):
        for dx in range(3):
            st = base + (dy - 1) * (base - 1) + (dx - 1)
            cols.append(x_frame[st:st + m_rows, :])
    cin = x_frame.shape[1]
    xcat = jnp.concatenate(cols, axis=1)
    wflat = w9.reshape(9 * cin, w9.shape[2])
    return jnp.dot(xcat, wflat, preferred_element_type=jnp.float32)


def _embed(y, base, cout):
    z = jnp.zeros((base, cout), dtype=jnp.float32)
    return jnp.concatenate([z, y, z], axis=0)


def _kernel_a(x_ref, w1_ref, b1_ref, w2_ref, b2_ref, m_ref, o_ref):
    x = x_ref[0]
    mask = m_ref[...]
    y1 = jnp.maximum(_conv9(x, w1_ref[...], M1, BASE1) + b1_ref[...], 0.) * mask
    f2 = _embed(y1, BASE1, 64)
    y2 = jnp.maximum(_conv9(f2, w2_ref[...], M1, BASE1) + b2_ref[...], 0.) * mask
    f3 = _embed(y2, BASE1, 64).reshape(H1, W1, 64)
    inter = f3[1:65, 1:65, :]
    t = inter.reshape(1024, 4, 64).max(axis=1)
    t = t.reshape(16, 4, 16, 64).max(axis=1)
    o_ref[0] = t.reshape(256, 64)


def _kernel_b(x_ref, w3_ref, b3_ref, w4_ref, b4_ref, w5_ref, b5_ref,
              w6_ref, b6_ref, m2_ref, m3_ref, o_ref):
    x = x_ref[0]
    mask2 = m2_ref[...]
    y3 = jnp.maximum(_conv9(x, w3_ref[...], M2, BASE2) + b3_ref[...], 0.) * mask2
    f4 = _embed(y3, BASE2, 128)
    y4 = jnp.maximum(_conv9(f4, w4_ref[...], M2, BASE2) + b4_ref[...], 0.) * mask2
    f5 = _embed(y4, BASE2, 128).reshape(H2, W2, 128)
    inter = f5[1:17, 1:17, :]
    t = inter.reshape(128, 2, 128).max(axis=1)
    t = t.reshape(8, 2, 8, 128).max(axis=1)
    zc = jnp.zeros((8, 1, 128), dtype=jnp.float32)
    zc7 = jnp.zeros((8, 7, 128), dtype=jnp.float32)
    t = jnp.concatenate([zc, t, zc7], axis=1)
    zr = jnp.zeros((1, 16, 128), dtype=jnp.float32)
    f6 = jnp.concatenate([zr, t, zr], axis=0).reshape(N3, 128)
    mask3 = m3_ref[...]
    y5 = jnp.maximum(_conv9(f6, w5_ref[...], M3, BASE3) + b5_ref[...], 0.) * mask3
    f7 = _embed(y5, BASE3, 512)
    y6 = jnp.maximum(_conv9(f7, w6_ref[...], M3, BASE3) + b6_ref[...], 0.) * mask3
    f8 = _embed(y6, BASE3, 512).reshape(H3, W3, 512)
    inter3 = f8[1:9, 1:9, :].reshape(64, 512)
    o_ref[0] = inter3.T


def _kernel_c(x_ref, pw1_ref, pb1_ref, pw2_ref, pb2_ref, pw3_ref, pb3_ref,
              o_ref, acc_ref):
    i = pl.program_id(0)

    @pl.when(i == 0)
    def _():
        acc_ref[...] = jnp.zeros_like(acc_ref)

    acc_ref[...] += jnp.dot(x_ref[...], pw1_ref[...],
                            preferred_element_type=jnp.float32)

    @pl.when(i == pl.num_programs(0) - 1)
    def _():
        h1 = jnp.maximum(acc_ref[...] + pb1_ref[...], 0.)
        h2 = jnp.maximum(jnp.dot(h1, pw2_ref[...],
                                 preferred_element_type=jnp.float32)
                         + pb2_ref[...], 0.)
        o_ref[...] = jnp.dot(h2, pw3_ref[...],
                             preferred_element_type=jnp.float32) + pb3_ref[...]


def _kernel_d(bb_ref, img_ref, py_ref, px_ref, chb_ref, hs_ref, hv_ref, o_ref):
    bb = bb_ref[0]
    ymin = bb[0:1, 0:1] - 1.0
    ymax = bb[0:1, 1:2] + 1.0
    xmin = bb[0:1, 2:3] - 1.0
    xmax = bb[0:1, 3:4] + 1.0
    my = jax.nn.sigmoid(py_ref[...] * (ymax - ymin) + ymin) * (H - 1)
    mx = jax.nn.sigmoid(px_ref[...] * (xmax - xmin) + xmin) * (W - 1)
    y0 = jnp.floor(my)
    x0 = jnp.floor(mx)
    hs = hs_ref[...]
    sp = jnp.maximum(hs, 0.) + jnp.log1p(jnp.exp(-jnp.abs(hs)))
    sig = sp + 1e-4
    imgT = img_ref[0]
    corners = [(0., 0.), (0., 1.), (1., 0.), (1., 1.)]
    es, gs = [], []
    for dy, dx in corners:
        yc = y0 + dy
        xc = x0 + dx
        d2 = (yc - my) ** 2 + (xc - mx) ** 2
        es.append(jnp.exp(-d2 / (2.0 * sig * sig)))
        yi = jnp.clip(yc, 0., H - 1.)
        xi = jnp.clip(xc, 0., W - 1.)
        rid = (chb_ref[...] + yi).astype(jnp.int32)
        oh = (jax.lax.broadcasted_iota(jnp.int32, (3 * H, L), 0)
              == rid).astype(jnp.float32)
        g = jnp.dot(imgT, oh, preferred_element_type=jnp.float32, precision=jax.lax.Precision.HIGHEST)
        ohx = (jax.lax.broadcasted_iota(jnp.int32, (W, L), 0)
               == xi.astype(jnp.int32)).astype(jnp.float32)
        gs.append(jnp.sum(g * ohx, axis=0, keepdims=True))
    denom = es[0] + es[1] + es[2] + es[3] + 1e-9
    point = (gs[0] * es[0] + gs[1] * es[1] + gs[2] * es[2] + gs[3] * es[3])
    point = point / denom * hv_ref[...]
    o_ref[0] = point


def _kernel_e(x_ref, hb_ref, w1_ref, b1_ref, w2_ref, b2_ref, o_ref):
    x = x_ref[...] + hb_ref[...]
    h1 = jnp.maximum(jnp.dot(x, w1_ref[...],
                             preferred_element_type=jnp.float32) + b1_ref[...], 0.)
    lg = jnp.dot(h1, w2_ref[...], preferred_element_type=jnp.float32) + b2_ref[...]
    m = jnp.max(lg, axis=1, keepdims=True)
    e = jnp.exp(lg - m)
    o_ref[...] = e / jnp.sum(e, axis=1, keepdims=True)


def _wt(cw, cin, cout):
    return cw.transpose(2, 3, 1, 0).reshape(9, cin, cout)


def kernel(image, cw1, cb1, cw2, cb2, cw3, cb3, cw4, cb4, cw5, cb5, cw6, cb6,
           pw1, pb1, pw2, pb2, pw3, pb3, h_sigmas, h_values, h_bias,
           lin1_w, lin1_b, lin2_w, lin2_b):
    f32 = jnp.float32
    b = image.shape[0]

    xp = jnp.pad(image.transpose(0, 2, 3, 1), ((0, 0), (1, 1), (1, 7), (0, 0)))
    xp = xp.reshape(b, N1, C)
    a_out = pl.pallas_call(
        _kernel_a,
        grid=(b,),
        in_specs=[
            pl.BlockSpec((1, N1, C), lambda i: (i, 0, 0)),
            pl.BlockSpec((9, C, 64), lambda i: (0, 0, 0)),
            pl.BlockSpec((1, 64), lambda i: (0, 0)),
            pl.BlockSpec((9, 64, 64), lambda i: (0, 0, 0)),
            pl.BlockSpec((1, 64), lambda i: (0, 0)),
            pl.BlockSpec((M1, 1), lambda i: (0, 0)),
        ],
        out_specs=pl.BlockSpec((1, 256, 64), lambda i: (i, 0, 0)),
        out_shape=jax.ShapeDtypeStruct((b, 256, 64), f32),
    )(xp, _wt(cw1, C, 64), cb1.reshape(1, 64), _wt(cw2, 64, 64),
      cb2.reshape(1, 64), jnp.asarray(_MASK1))

    xb = jnp.pad(a_out.reshape(b, 16, 16, 64), ((0, 0), (1, 1), (1, 7), (0, 0)))
    xb = xb.reshape(b, N2, 64)
    b_out = pl.pallas_call(
        _kernel_b,
        grid=(b,),
        in_specs=[
            pl.BlockSpec((1, N2, 64), lambda i: (i, 0, 0)),
            pl.BlockSpec((9, 64, 128), lambda i: (0, 0, 0)),
            pl.BlockSpec((1, 128), lambda i: (0, 0)),
            pl.BlockSpec((9, 128, 128), lambda i: (0, 0, 0)),
            pl.BlockSpec((1, 128), lambda i: (0, 0)),
            pl.BlockSpec((9, 128, 512), lambda i: (0, 0, 0)),
            pl.BlockSpec((1, 512), lambda i: (0, 0)),
            pl.BlockSpec((9, 512, 512), lambda i: (0, 0, 0)),
            pl.BlockSpec((1, 512), lambda i: (0, 0)),
            pl.BlockSpec((M2, 1), lambda i: (0, 0)),
            pl.BlockSpec((M3, 1), lambda i: (0, 0)),
        ],
        out_specs=pl.BlockSpec((1, 512, 64), lambda i: (i, 0, 0)),
        out_shape=jax.ShapeDtypeStruct((b, 512, 64), f32),
    )(xb, _wt(cw3, 64, 128), cb3.reshape(1, 128), _wt(cw4, 128, 128),
      cb4.reshape(1, 128), _wt(cw5, 128, 512), cb5.reshape(1, 512),
      _wt(cw6, 512, 512), cb6.reshape(1, 512),
      jnp.asarray(_MASK2), jnp.asarray(_MASK3))

    flat = b_out.reshape(b, 32768)
    kc = 16
    blk = 32768 // kc
    bbox = pl.pallas_call(
        _kernel_c,
        grid=(kc,),
        in_specs=[
            pl.BlockSpec((b, blk), lambda i: (0, i)),
            pl.BlockSpec((blk, 512), lambda i: (i, 0)),
            pl.BlockSpec((1, 512), lambda i: (0, 0)),
            pl.BlockSpec((512, 512), lambda i: (0, 0)),
            pl.BlockSpec((1, 512), lambda i: (0, 0)),
            pl.BlockSpec((512, 4), lambda i: (0, 0)),
            pl.BlockSpec((1, 4), lambda i: (0, 0)),
        ],
        out_specs=pl.BlockSpec((b, 4), lambda i: (0, 0)),
        out_shape=jax.ShapeDtypeStruct((b, 4), f32),
        scratch_shapes=[pltpu.VMEM((b, 512), f32)],
        compiler_params=pltpu.CompilerParams(
            dimension_semantics=("arbitrary",)),
    )(flat, pw1, pb1.reshape(1, 512), pw2, pb2.reshape(1, 512),
      pw3, pb3.reshape(1, 4))

    imgT = image.transpose(0, 3, 1, 2).reshape(b, W, C * H)
    point = pl.pallas_call(
        _kernel_d,
        grid=(b,),
        in_specs=[
            pl.BlockSpec((1, 1, 4), lambda i: (i, 0, 0)),
            pl.BlockSpec((1, W, C * H), lambda i: (i, 0, 0)),
            pl.BlockSpec((1, L), lambda i: (0, 0)),
            pl.BlockSpec((1, L), lambda i: (0, 0)),
            pl.BlockSpec((1, L), lambda i: (0, 0)),
            pl.BlockSpec((1, L), lambda i: (0, 0)),
            pl.BlockSpec((1, L), lambda i: (0, 0)),
        ],
        out_specs=pl.BlockSpec((1, 1, L), lambda i: (i, 0, 0)),
        out_shape=jax.ShapeDtypeStruct((b, 1, L), f32),
    )(bbox.reshape(b, 1, 4), imgT, jnp.asarray(_PY), jnp.asarray(_PX), jnp.asarray(_CHB),
      (h_sigmas * 0.1 + 0.01).reshape(1, L), h_values.reshape(1, L))

    flat_g = point.reshape(b, L)[:, jnp.asarray(_PERM)]
    probs = pl.pallas_call(
        _kernel_e,
        out_shape=jax.ShapeDtypeStruct((b, NUM_CLASSES), f32),
    )(flat_g, h_bias.reshape(1, L), lin1_w, lin1_b.reshape(1, HIDDEN),
      lin2_w, lin2_b.reshape(1, NUM_CLASSES))
    return probs
